# baseline (speedup 1.0000x reference)
import functools

import jax
import jax.numpy as jnp
from jax import lax
from jax.experimental import pallas as pl
from jax.experimental.pallas import tpu as pltpu
from jax.experimental.pallas import tpu_sc as plsc

N_NODES = 10000
D_FEAT = 128
D_HID = 20
DP = 24

NC = 2
NS = 16
NW = NC * NS
NPAD = 10240
N_DUMMY = NPAD - N_NODES
CHUNK = 1024
E_PER_W = 10240
N_CHUNKS = E_PER_W // CHUNK
EPAD = NW * E_PER_W

_MESH = plsc.VectorSubcoreMesh(core_axis_name="c", subcore_axis_name="s")
_SC_PARAMS = pltpu.CompilerParams(use_tc_tiling_on_sc=False,
                                  needs_layout_passes=False)


@functools.partial(
    pl.kernel,
    mesh=_MESH,
    out_type=jax.ShapeDtypeStruct((NC, NPAD), jnp.float32),
    compiler_params=_SC_PARAMS,
    scratch_types=[
        pltpu.VMEM((CHUNK,), jnp.int32),
        pltpu.VMEM((CHUNK,), jnp.int32),
        pltpu.VMEM((CHUNK,), jnp.float32),
        pltpu.VMEM_SHARED((NPAD,), jnp.float32),
        pltpu.SemaphoreType.DMA,
        pltpu.SemaphoreType.DMA,
    ],
)
def _sc_degree(dst_hbm, zeros_hbm, deg_out, idx0_v, idx1_v, ones_v, acc_sh,
               sem0, sem1):
    c = lax.axis_index("c")
    s = lax.axis_index("s")
    w = c * NS + s
    for i in range(CHUNK // 16):
        ones_v[pl.ds(i * 16, 16)] = jnp.ones((16,), jnp.float32)

    @pl.when(s == 0)
    def _():
        pltpu.sync_copy(zeros_hbm, acc_sh)

    plsc.subcore_barrier()

    idx_bufs = (idx0_v, idx1_v)
    sems = (sem0, sem1)
    pend = [None, None]
    for j in range(N_CHUNKS):
        b = j % 2
        base = w * E_PER_W + j * CHUNK
        if pend[b] is not None:
            pend[b].wait()
            pend[b] = None
        pltpu.sync_copy(dst_hbm.at[pl.ds(base, CHUNK)], idx_bufs[b])
        pend[b] = pltpu.async_copy(ones_v, acc_sh.at[idx_bufs[b]], sems[b],
                                   add=True)
    for b in range(2):
        if pend[b] is not None:
            pend[b].wait()
    plsc.subcore_barrier()
    rows = NPAD // NS
    pltpu.sync_copy(acc_sh.at[pl.ds(s * rows, rows)],
                    deg_out.at[c, pl.ds(s * rows, rows)])


def _make_sc_agg(with_g):
    out_type = (
        jax.ShapeDtypeStruct((NC, NPAD, DP), jnp.float32),
        jax.ShapeDtypeStruct((NC, NPAD), jnp.float32),
    ) if with_g else jax.ShapeDtypeStruct((NC, NPAD, DP), jnp.float32)
    scratch = [
        pltpu.VMEM((CHUNK,), jnp.int32),
        pltpu.VMEM((CHUNK,), jnp.int32),
        pltpu.VMEM((CHUNK,), jnp.int32),
        pltpu.VMEM((CHUNK,), jnp.int32),
        pltpu.VMEM((CHUNK, DP), jnp.float32),
        pltpu.VMEM((CHUNK, DP), jnp.float32),
        pltpu.VMEM((CHUNK,), jnp.float32),
        pltpu.VMEM((CHUNK,), jnp.float32),
        pltpu.VMEM((NPAD,), jnp.float32),
        pltpu.VMEM_SHARED((NPAD, DP), jnp.float32),
        pltpu.VMEM_SHARED((NPAD,), jnp.float32),
        pltpu.SemaphoreType.DMA,
        pltpu.SemaphoreType.DMA,
        pltpu.SemaphoreType.DMA,
        pltpu.SemaphoreType.DMA,
    ]

    @functools.partial(pl.kernel, mesh=_MESH, out_type=out_type,
                       compiler_params=_SC_PARAMS, scratch_types=scratch)
    def body(u_hbm, dis_hbm, src_hbm, dst_hbm, zeros2_hbm, zeros1_hbm,
             *rest):
        if with_g:
            (agg_out, g_out, src0, src1, dst0, dst1, rows0, rows1, dval0,
             dval1, dis_l, acc_sh, g_sh, gsem0, gsem1, ssem0, ssem1) = rest
        else:
            (agg_out, src0, src1, dst0, dst1, rows0, rows1, dval0,
             dval1, dis_l, acc_sh, g_sh, gsem0, gsem1, ssem0, ssem1) = rest
        c = lax.axis_index("c")
        s = lax.axis_index("s")
        w = c * NS + s

        @pl.when(s == 0)
        def _():
            pltpu.sync_copy(zeros2_hbm, acc_sh)
            if with_g:
                pltpu.sync_copy(zeros1_hbm, g_sh)

        if with_g:
            pltpu.sync_copy(dis_hbm, dis_l)
        plsc.subcore_barrier()

        srcb = (src0, src1)
        dstb = (dst0, dst1)
        rowsb = (rows0, rows1)
        dvalb = (dval0, dval1)
        gsems = (gsem0, gsem1)
        ssems = (ssem0, ssem1)
        gpend = [[], []]
        spend = [[], []]

        def load_and_gather(j):
            b = j % 2
            base = w * E_PER_W + j * CHUNK
            pltpu.sync_copy(src_hbm.at[pl.ds(base, CHUNK)], srcb[b])
            pltpu.sync_copy(dst_hbm.at[pl.ds(base, CHUNK)], dstb[b])
            gpend[b].append(
                pltpu.async_copy(u_hbm.at[srcb[b]], rowsb[b], gsems[b]))
            if with_g:
                for k in range(CHUNK // 16):
                    idx16 = dstb[b][pl.ds(k * 16, 16)]
                    dvalb[b][pl.ds(k * 16, 16)] = plsc.load_gather(
                        dis_l, [idx16])

        load_and_gather(0)
        for j in range(N_CHUNKS):
            b = j % 2
            nb = 1 - b
            if j + 1 < N_CHUNKS:
                for d in spend[nb]:
                    d.wait()
                spend[nb] = []
                load_and_gather(j + 1)
            for d in gpend[b]:
                d.wait()
            gpend[b] = []
            spend[b].append(
                pltpu.async_copy(rowsb[b], acc_sh.at[dstb[b]], ssems[b],
                                 add=True))
            if with_g:
                spend[b].append(
                    pltpu.async_copy(dvalb[b], g_sh.at[srcb[b]], ssems[b],
                                     add=True))
        for b in range(2):
            for d in spend[b]:
                d.wait()
        plsc.subcore_barrier()
        rows = NPAD // NS
        pltpu.sync_copy(acc_sh.at[pl.ds(s * rows, rows)],
                        agg_out.at[c, pl.ds(s * rows, rows)])
        if with_g:
            pltpu.sync_copy(g_sh.at[pl.ds(s * rows, rows)],
                            g_out.at[c, pl.ds(s * rows, rows)])

    return body


_sc_agg_g = _make_sc_agg(True)

_ROWS_T = NPAD // NS


@functools.partial(
    pl.kernel,
    mesh=_MESH,
    out_type=(
        jax.ShapeDtypeStruct((NC, NPAD, DP), jnp.float32),
        jax.ShapeDtypeStruct((NC * NPAD, DP), jnp.float32),
    ),
    compiler_params=_SC_PARAMS,
    scratch_types=[
        pltpu.VMEM((CHUNK,), jnp.int32),
        pltpu.VMEM((CHUNK,), jnp.int32),
        pltpu.VMEM((CHUNK,), jnp.int32),
        pltpu.VMEM((CHUNK,), jnp.int32),
        pltpu.VMEM((CHUNK, DP), jnp.float32),
        pltpu.VMEM((CHUNK, DP), jnp.float32),
        pltpu.VMEM((_ROWS_T, DP), jnp.float32),
        pltpu.VMEM((_ROWS_T, DP), jnp.float32),
        pltpu.VMEM((_ROWS_T, DP), jnp.float32),
        pltpu.VMEM((_ROWS_T, DP), jnp.float32),
        pltpu.VMEM((_ROWS_T,), jnp.float32),
        pltpu.VMEM((DP,), jnp.float32),
        pltpu.VMEM_SHARED((NPAD, DP), jnp.float32),
        pltpu.SemaphoreType.DMA,
        pltpu.SemaphoreType.DMA,
        pltpu.SemaphoreType.DMA,
        pltpu.SemaphoreType.DMA,
    ],
)
def _sc_agg2(p_hbm, u1_hbm, dis_hbm, b1_hbm, src_hbm, dst_hbm, zeros2_hbm,
             agg_out, u2sc_out, src0, src1, dst0, dst1, rows0, rows1,
             pbuf0, pbuf1, ubuf, u2buf, dis_s, b1_l, acc_sh,
             gsem0, gsem1, ssem0, ssem1):
    c = lax.axis_index("c")
    s = lax.axis_index("s")
    w = c * NS + s
    r0 = s * _ROWS_T

    @pl.when(s == 0)
    def _():
        pltpu.sync_copy(zeros2_hbm, acc_sh)

    pltpu.sync_copy(p_hbm.at[0, pl.ds(r0, _ROWS_T)], pbuf0)
    pltpu.sync_copy(p_hbm.at[1, pl.ds(r0, _ROWS_T)], pbuf1)
    pltpu.sync_copy(u1_hbm.at[pl.ds(r0, _ROWS_T)], ubuf)
    pltpu.sync_copy(dis_hbm.at[pl.ds(r0, _ROWS_T)], dis_s)
    pltpu.sync_copy(b1_hbm, b1_l)

    def build(k, carry):
        pos = lax.iota(jnp.int32, 16) + k * 16
        ri = pos // DP
        ci = pos - ri * DP
        d = plsc.load_gather(dis_s, [ri])
        bb = plsc.load_gather(b1_l, [ci])
        agg = (plsc.load_gather(pbuf0, [ri, ci])
               + plsc.load_gather(pbuf1, [ri, ci])
               + plsc.load_gather(ubuf, [ri, ci]))
        val = d * jnp.maximum(d * agg + bb, 0.0)
        plsc.store_scatter(u2buf, [ri, ci], val)
        return carry

    lax.fori_loop(0, _ROWS_T * DP // 16, build, 0)
    pltpu.sync_copy(u2buf, u2sc_out.at[pl.ds(c * NPAD + r0, _ROWS_T)])
    plsc.subcore_barrier()

    srcb = (src0, src1)
    dstb = (dst0, dst1)
    rowsb = (rows0, rows1)
    gsems = (gsem0, gsem1)
    ssems = (ssem0, ssem1)
    gpend = [[], []]
    spend = [[], []]

    def load_and_gather(j):
        b = j % 2
        base = w * E_PER_W + j * CHUNK
        pltpu.sync_copy(src_hbm.at[pl.ds(base, CHUNK)], srcb[b])
        pltpu.sync_copy(dst_hbm.at[pl.ds(base, CHUNK)], dstb[b])

        def off(k, carry):
            sl = pl.ds(k * 16, 16)
            srcb[b][sl] = srcb[b][sl] + c * NPAD
            return carry

        lax.fori_loop(0, CHUNK // 16, off, 0)
        gpend[b].append(
            pltpu.async_copy(u2sc_out.at[srcb[b]], rowsb[b], gsems[b]))

    load_and_gather(0)
    for j in range(N_CHUNKS):
        b = j % 2
        nb = 1 - b
        if j + 1 < N_CHUNKS:
            for d in spend[nb]:
                d.wait()
            spend[nb] = []
            load_and_gather(j + 1)
        for d in gpend[b]:
            d.wait()
        gpend[b] = []
        spend[b].append(
            pltpu.async_copy(rowsb[b], acc_sh.at[dstb[b]], ssems[b],
                             add=True))
    for b in range(2):
        for d in spend[b]:
            d.wait()
    plsc.subcore_barrier()
    pltpu.sync_copy(acc_sh.at[pl.ds(r0, _ROWS_T)],
                    agg_out.at[c, pl.ds(r0, _ROWS_T)])


def _dense1_body(deg_ref, x_ref, w1_ref, dis_ref, u1_ref):
    deg = deg_ref[0, :] + deg_ref[1, :] + 1.0
    dis = lax.rsqrt(deg)
    dis_ref[:] = dis
    h = jnp.dot(x_ref[:], w1_ref[:], preferred_element_type=jnp.float32)
    u1_ref[:, :] = h * dis[:, None]


def _dense2_body(agg_ref, u1_ref, dis_ref, b1_ref, u2_ref):
    dis = dis_ref[:]
    agg = agg_ref[0] + agg_ref[1] + u1_ref[:, :]
    out1 = jnp.maximum(agg * dis[:, None] + b1_ref[:, :], 0.0)
    u2_ref[:, :] = out1 * dis[:, None]


def _dense3_body(agg_ref, u2_ref, dis_ref, g_ref, w2_ref, b2_ref, w3_ref,
                 b3_ref, out_ref):
    dis = dis_ref[:]
    v = (agg_ref[0] + agg_ref[1] + u2_ref[0:NPAD, :]) * dis[:, None]
    out2 = jnp.maximum(
        jnp.dot(v, w2_ref[:], preferred_element_type=jnp.float32)
        + b2_ref[:, :], 0.0)
    h3 = jnp.dot(out2, w3_ref[:], preferred_element_type=jnp.float32)
    g = g_ref[0, :] + g_ref[1, :]
    cvec = dis * (g + dis)
    node = lax.broadcasted_iota(jnp.int32, (NPAD, 1), 0)
    cvec = jnp.where(node[:, 0] < N_NODES, cvec, 0.0)
    total = jnp.sum(cvec * h3[:, 0]) + N_NODES * b3_ref[0, 0]
    out_ref[:, :] = total[None, None]


def kernel(feature_mtx, adj_mtx_coo, W1, b1, W2, b2, W3, b3):
    src = adj_mtx_coo[:, 0].astype(jnp.int32)
    dst = adj_mtx_coo[:, 1].astype(jnp.int32)
    n_pad_e = EPAD - src.shape[0]
    pad_idx = (jnp.arange(n_pad_e, dtype=jnp.int32) % N_DUMMY) + N_NODES
    src_p = jnp.concatenate([src, pad_idx])
    dst_p = jnp.concatenate([dst, pad_idx])
    x_pad = jnp.pad(feature_mtx, ((0, N_DUMMY), (0, 0)))
    w1_pad = jnp.pad(W1, ((0, 0), (0, DP - D_HID)))
    b1_pad = jnp.pad(b1.reshape(1, D_HID), ((0, 0), (0, DP - D_HID)))
    w2_pad = jnp.pad(W2, ((0, DP - D_HID), (0, 0)))
    zeros2 = jnp.zeros((NPAD, DP), jnp.float32)
    zeros1 = jnp.zeros((NPAD,), jnp.float32)

    deg_part = _sc_degree(dst_p, zeros1)

    dis, u1 = pl.pallas_call(
        _dense1_body,
        out_shape=[
            jax.ShapeDtypeStruct((NPAD,), jnp.float32),
            jax.ShapeDtypeStruct((NPAD, DP), jnp.float32),
        ],
    )(deg_part, x_pad, w1_pad)

    agg1_part, g_part = _sc_agg_g(u1, dis, src_p, dst_p, zeros2, zeros1)

    agg2_part, u2sc = _sc_agg2(agg1_part, u1, dis, b1_pad.reshape(DP),
                               src_p, dst_p, zeros2)

    out = pl.pallas_call(
        _dense3_body,
        out_shape=jax.ShapeDtypeStruct((1, 1), jnp.float32),
    )(agg2_part, u2sc, dis, g_part, w2_pad, b2.reshape(1, D_FEAT), W3,
      b3.reshape(1, 1))
    return out.reshape(1)


# speedup vs baseline: 1.0557x; 1.0557x over previous
import functools

import jax
import jax.numpy as jnp
from jax import lax
from jax.experimental import pallas as pl
from jax.experimental.pallas import tpu as pltpu
from jax.experimental.pallas import tpu_sc as plsc

N_NODES = 10000
D_FEAT = 128
D_HID = 20
DP = 24

NC = 2
NS = 16
NW = NC * NS
NPAD = 10240
N_DUMMY = NPAD - N_NODES
CHUNK = 1280
E_PER_W = 10240
N_CHUNKS = E_PER_W // CHUNK
EPAD = NW * E_PER_W

_MESH = plsc.VectorSubcoreMesh(core_axis_name="c", subcore_axis_name="s")
_SC_PARAMS = pltpu.CompilerParams(use_tc_tiling_on_sc=False,
                                  needs_layout_passes=False)


@functools.partial(
    pl.kernel,
    mesh=_MESH,
    out_type=jax.ShapeDtypeStruct((NC, NPAD), jnp.float32),
    compiler_params=_SC_PARAMS,
    scratch_types=[
        pltpu.VMEM((CHUNK,), jnp.int32),
        pltpu.VMEM((CHUNK,), jnp.int32),
        pltpu.VMEM((CHUNK,), jnp.float32),
        pltpu.VMEM_SHARED((NPAD,), jnp.float32),
        pltpu.SemaphoreType.DMA,
        pltpu.SemaphoreType.DMA,
    ],
)
def _sc_degree(dst_hbm, zeros_hbm, deg_out, idx0_v, idx1_v, ones_v, acc_sh,
               sem0, sem1):
    c = lax.axis_index("c")
    s = lax.axis_index("s")
    w = c * NS + s
    for i in range(CHUNK // 16):
        ones_v[pl.ds(i * 16, 16)] = jnp.ones((16,), jnp.float32)

    @pl.when(s == 0)
    def _():
        pltpu.sync_copy(zeros_hbm, acc_sh)

    plsc.subcore_barrier()

    idx_bufs = (idx0_v, idx1_v)
    sems = (sem0, sem1)
    pend = [None, None]
    for j in range(N_CHUNKS):
        b = j % 2
        base = w * E_PER_W + j * CHUNK
        if pend[b] is not None:
            pend[b].wait()
            pend[b] = None
        pltpu.sync_copy(dst_hbm.at[pl.ds(base, CHUNK)], idx_bufs[b])
        pend[b] = pltpu.async_copy(ones_v, acc_sh.at[idx_bufs[b]], sems[b],
                                   add=True)
    for b in range(2):
        if pend[b] is not None:
            pend[b].wait()
    plsc.subcore_barrier()
    rows = NPAD // NS
    pltpu.sync_copy(acc_sh.at[pl.ds(s * rows, rows)],
                    deg_out.at[c, pl.ds(s * rows, rows)])


def _make_sc_agg(with_g):
    out_type = (
        jax.ShapeDtypeStruct((NC, NPAD, DP), jnp.float32),
        jax.ShapeDtypeStruct((NC, NPAD), jnp.float32),
    ) if with_g else jax.ShapeDtypeStruct((NC, NPAD, DP), jnp.float32)
    scratch = [
        pltpu.VMEM((CHUNK,), jnp.int32),
        pltpu.VMEM((CHUNK,), jnp.int32),
        pltpu.VMEM((CHUNK,), jnp.int32),
        pltpu.VMEM((CHUNK,), jnp.int32),
        pltpu.VMEM((CHUNK, DP), jnp.float32),
        pltpu.VMEM((CHUNK, DP), jnp.float32),
        pltpu.VMEM((CHUNK,), jnp.float32),
        pltpu.VMEM((CHUNK,), jnp.float32),
        pltpu.VMEM((NPAD,), jnp.float32),
        pltpu.VMEM_SHARED((NPAD, DP), jnp.float32),
        pltpu.VMEM_SHARED((NPAD,), jnp.float32),
        pltpu.SemaphoreType.DMA,
        pltpu.SemaphoreType.DMA,
        pltpu.SemaphoreType.DMA,
        pltpu.SemaphoreType.DMA,
    ]

    @functools.partial(pl.kernel, mesh=_MESH, out_type=out_type,
                       compiler_params=_SC_PARAMS, scratch_types=scratch)
    def body(u_hbm, dis_hbm, src_hbm, dst_hbm, zeros2_hbm, zeros1_hbm,
             *rest):
        if with_g:
            (agg_out, g_out, src0, src1, dst0, dst1, rows0, rows1, dval0,
             dval1, dis_l, acc_sh, g_sh, gsem0, gsem1, ssem0, ssem1) = rest
        else:
            (agg_out, src0, src1, dst0, dst1, rows0, rows1, dval0,
             dval1, dis_l, acc_sh, g_sh, gsem0, gsem1, ssem0, ssem1) = rest
        c = lax.axis_index("c")
        s = lax.axis_index("s")
        w = c * NS + s

        @pl.when(s == 0)
        def _():
            pltpu.sync_copy(zeros2_hbm, acc_sh)
            if with_g:
                pltpu.sync_copy(zeros1_hbm, g_sh)

        if with_g:
            pltpu.sync_copy(dis_hbm, dis_l)
        plsc.subcore_barrier()

        srcb = (src0, src1)
        dstb = (dst0, dst1)
        rowsb = (rows0, rows1)
        dvalb = (dval0, dval1)
        gsems = (gsem0, gsem1)
        ssems = (ssem0, ssem1)
        gpend = [[], []]
        spend = [[], []]

        def load_and_gather(j):
            b = j % 2
            base = w * E_PER_W + j * CHUNK
            pltpu.sync_copy(src_hbm.at[pl.ds(base, CHUNK)], srcb[b])
            pltpu.sync_copy(dst_hbm.at[pl.ds(base, CHUNK)], dstb[b])
            gpend[b].append(
                pltpu.async_copy(u_hbm.at[srcb[b]], rowsb[b], gsems[b]))
            if with_g:
                for k in range(CHUNK // 16):
                    idx16 = dstb[b][pl.ds(k * 16, 16)]
                    dvalb[b][pl.ds(k * 16, 16)] = plsc.load_gather(
                        dis_l, [idx16])

        load_and_gather(0)
        for j in range(N_CHUNKS):
            b = j % 2
            nb = 1 - b
            if j + 1 < N_CHUNKS:
                for d in spend[nb]:
                    d.wait()
                spend[nb] = []
                load_and_gather(j + 1)
            for d in gpend[b]:
                d.wait()
            gpend[b] = []
            spend[b].append(
                pltpu.async_copy(rowsb[b], acc_sh.at[dstb[b]], ssems[b],
                                 add=True))
            if with_g:
                spend[b].append(
                    pltpu.async_copy(dvalb[b], g_sh.at[srcb[b]], ssems[b],
                                     add=True))
        for b in range(2):
            for d in spend[b]:
                d.wait()
        plsc.subcore_barrier()
        rows = NPAD // NS
        pltpu.sync_copy(acc_sh.at[pl.ds(s * rows, rows)],
                        agg_out.at[c, pl.ds(s * rows, rows)])
        if with_g:
            pltpu.sync_copy(g_sh.at[pl.ds(s * rows, rows)],
                            g_out.at[c, pl.ds(s * rows, rows)])

    return body


_sc_agg_g = _make_sc_agg(True)
_sc_agg = _make_sc_agg(False)


def _dense1_body(deg_ref, x_ref, w1_ref, dis_ref, u1_ref):
    deg = deg_ref[0, :] + deg_ref[1, :] + 1.0
    dis = lax.rsqrt(deg)
    dis_ref[:] = dis
    h = jnp.dot(x_ref[:], w1_ref[:], preferred_element_type=jnp.float32)
    u1_ref[:, :] = h * dis[:, None]


def _dense2_body(agg_ref, u1_ref, dis_ref, b1_ref, u2_ref):
    dis = dis_ref[:]
    agg = agg_ref[0] + agg_ref[1] + u1_ref[:, :]
    out1 = jnp.maximum(agg * dis[:, None] + b1_ref[:, :], 0.0)
    u2_ref[:, :] = out1 * dis[:, None]


def _dense3_body(agg_ref, u2_ref, dis_ref, g_ref, w2_ref, b2_ref, w3_ref,
                 b3_ref, out_ref):
    dis = dis_ref[:]
    v = (agg_ref[0] + agg_ref[1] + u2_ref[:, :]) * dis[:, None]
    out2 = jnp.maximum(
        jnp.dot(v, w2_ref[:], preferred_element_type=jnp.float32)
        + b2_ref[:, :], 0.0)
    h3 = jnp.dot(out2, w3_ref[:], preferred_element_type=jnp.float32)
    g = g_ref[0, :] + g_ref[1, :]
    cvec = dis * (g + dis)
    node = lax.broadcasted_iota(jnp.int32, (NPAD, 1), 0)
    cvec = jnp.where(node[:, 0] < N_NODES, cvec, 0.0)
    total = jnp.sum(cvec * h3[:, 0]) + N_NODES * b3_ref[0, 0]
    out_ref[:, :] = total[None, None]


def kernel(feature_mtx, adj_mtx_coo, W1, b1, W2, b2, W3, b3):
    src = adj_mtx_coo[:, 0].astype(jnp.int32)
    dst = adj_mtx_coo[:, 1].astype(jnp.int32)
    n_pad_e = EPAD - src.shape[0]
    pad_idx = (jnp.arange(n_pad_e, dtype=jnp.int32) % N_DUMMY) + N_NODES
    src_p = jnp.concatenate([src, pad_idx])
    dst_p = jnp.concatenate([dst, pad_idx])
    x_pad = jnp.pad(feature_mtx, ((0, N_DUMMY), (0, 0)))
    w1_pad = jnp.pad(W1, ((0, 0), (0, DP - D_HID)))
    b1_pad = jnp.pad(b1.reshape(1, D_HID), ((0, 0), (0, DP - D_HID)))
    w2_pad = jnp.pad(W2, ((0, DP - D_HID), (0, 0)))
    zeros2 = jnp.zeros((NPAD, DP), jnp.float32)
    zeros1 = jnp.zeros((NPAD,), jnp.float32)

    deg_part = _sc_degree(dst_p, zeros1)

    dis, u1 = pl.pallas_call(
        _dense1_body,
        out_shape=[
            jax.ShapeDtypeStruct((NPAD,), jnp.float32),
            jax.ShapeDtypeStruct((NPAD, DP), jnp.float32),
        ],
    )(deg_part, x_pad, w1_pad)

    agg1_part, g_part = _sc_agg_g(u1, dis, src_p, dst_p, zeros2, zeros1)

    u2 = pl.pallas_call(
        _dense2_body,
        out_shape=jax.ShapeDtypeStruct((NPAD, DP), jnp.float32),
    )(agg1_part, u1, dis, b1_pad)

    agg2_part = _sc_agg(u2, dis, src_p, dst_p, zeros2, zeros1)

    out = pl.pallas_call(
        _dense3_body,
        out_shape=jax.ShapeDtypeStruct((1, 1), jnp.float32),
    )(agg2_part, u2, dis, g_part, w2_pad, b2.reshape(1, D_FEAT), W3,
      b3.reshape(1, 1))
    return out.reshape(1)
